# baseline (speedup 1.0000x reference)
import functools

import jax
import jax.numpy as jnp
from jax import lax
from jax.experimental import pallas as pl
from jax.experimental.pallas import tpu as pltpu
from jax.experimental.pallas import tpu_sc as plsc

N = 10000
D = 128
H = 8
C = 16
E = 320000

NW = 32
CH = 80
NCHUNK0 = 150
NCHUNK1 = 102
NROWS = 16 * (NCHUNK0 + NCHUNK1)
EPAD = NROWS * CH
NP = 10016
RPT = NP // 16
NDS = 3
NIS = 6
R = 1000


def _leaky(x):
    return jnp.maximum(x, 0.0) + 0.2 * jnp.minimum(x, 0.0)


def _pre_body(x_ref, pW_ref, pb_ref, W_ref, ats_ref, atd_ref, GT_ref,
              h_ref, xw_ref, A_ref, B_ref):
    xb = x_ref[...]
    h = jnp.dot(xb, pW_ref[...], preferred_element_type=jnp.float32) + pb_ref[...]
    xw = jnp.dot(h, W_ref[...], preferred_element_type=jnp.float32)
    a_s = jnp.dot(xw * ats_ref[...], GT_ref[...], preferred_element_type=jnp.float32)
    a_d = jnp.dot(xw * atd_ref[...], GT_ref[...], preferred_element_type=jnp.float32)
    h_ref[...] = h
    xw_ref[...] = xw
    A_ref[...] = jnp.concatenate([a_s, a_d], axis=1)
    B_ref[...] = jnp.concatenate([a_d, a_s], axis=1)


def _combine(m0_ref, m1_ref, d0_ref, d1_ref, A_ref, xw_ref, hres_ref,
             bias_ref, g_ref, b_ref, G_ref):
    m = m0_ref[...] + m1_ref[...]
    d16 = d0_ref[...] + d1_ref[...]
    Ab = A_ref[...]
    es = jnp.exp(_leaky(Ab[:, :H] + Ab[:, H:]))
    xwb = xw_ref[...]
    G = G_ref[...]
    dtot = d16[:, :H] + es + 1e-16
    mtot = m + jnp.dot(es, G, preferred_element_type=jnp.float32) * xwb
    out = mtot / jnp.dot(dtot, G, preferred_element_type=jnp.float32)
    y = jnp.maximum(out + bias_ref[...], 0.0) + hres_ref[...]
    mu = jnp.mean(y, axis=1, keepdims=True)
    yc = y - mu
    var = jnp.mean(yc * yc, axis=1, keepdims=True)
    return yc * lax.rsqrt(var + 1e-5) * g_ref[...] + b_ref[...]


def _mid_body(m0_ref, m1_ref, d0_ref, d1_ref, A_ref, xw_ref, hres_ref,
              bias_ref, g_ref, b_ref, G_ref, W_ref, ats_ref, atd_ref, GT_ref,
              h_ref, xw_out_ref, A_out_ref, B_out_ref):
    hn = _combine(m0_ref, m1_ref, d0_ref, d1_ref, A_ref, xw_ref, hres_ref,
                  bias_ref, g_ref, b_ref, G_ref)
    xwn = jnp.dot(hn, W_ref[...], preferred_element_type=jnp.float32)
    a_s = jnp.dot(xwn * ats_ref[...], GT_ref[...], preferred_element_type=jnp.float32)
    a_d = jnp.dot(xwn * atd_ref[...], GT_ref[...], preferred_element_type=jnp.float32)
    h_ref[...] = hn
    xw_out_ref[...] = xwn
    A_out_ref[...] = jnp.concatenate([a_s, a_d], axis=1)
    B_out_ref[...] = jnp.concatenate([a_d, a_s], axis=1)


def _post_body(m0_ref, m1_ref, d0_ref, d1_ref, A_ref, xw_ref, hres_ref,
               bias_ref, g_ref, b_ref, G_ref,
               W1_ref, b1_ref, W2_ref, b2_ref, W3_ref, b3_ref, out_ref):
    hn = _combine(m0_ref, m1_ref, d0_ref, d1_ref, A_ref, xw_ref, hres_ref,
                  bias_ref, g_ref, b_ref, G_ref)
    y1 = jnp.maximum(jnp.dot(hn, W1_ref[...], preferred_element_type=jnp.float32)
                     + b1_ref[...], 0.0)
    y2 = jnp.maximum(jnp.dot(y1, W2_ref[...], preferred_element_type=jnp.float32)
                     + b2_ref[...], 0.0)
    out_ref[...] = (jnp.dot(y2, W3_ref[...], preferred_element_type=jnp.float32)
                    + b3_ref[...])


def _row_spec(cols):
    return pl.BlockSpec((R, cols), lambda i: (i, 0))


def _full_spec(shape):
    nd = len(shape)
    return pl.BlockSpec(shape, lambda i: (0,) * nd)


def _sc_edge_body(sd_hbm, A_hbm, B_hbm, xw_hbm, zm_hbm, zd_hbm,
                  msg0, msg1, den0, den1, *scr):
    sdidx = scr[0:NIS]
    va = scr[NIS:NIS + NDS]
    vb = scr[NIS + NDS:NIS + 2 * NDS]
    xwb = scr[NIS + 2 * NDS:NIS + 3 * NDS]
    semi = scr[NIS + 3 * NDS:2 * NIS + 3 * NDS]
    semg = scr[2 * NIS + 3 * NDS:2 * NIS + 4 * NDS]
    sems = scr[2 * NIS + 4 * NDS:2 * NIS + 5 * NDS]
    sh_msg, sh_den = scr[-2], scr[-1]

    c = lax.axis_index("c")
    s = lax.axis_index("s")
    rowbase = jnp.where(c == 0, s * NCHUNK0, 16 * NCHUNK0 + s * NCHUNK1)

    def fire_idx(j, k):
        pltpu.async_copy(sd_hbm.at[rowbase + j], sdidx[k], semi[k])

    def wait_idx(k):
        pltpu.make_async_copy(sd_hbm.at[rowbase], sdidx[k], semi[k]).wait()

    def fire_gather(k, d):
        pltpu.async_copy(A_hbm.at[sdidx[k].at[0]], va[d], semg[d])
        pltpu.async_copy(B_hbm.at[sdidx[k].at[1]], vb[d], semg[d])
        pltpu.async_copy(xw_hbm.at[sdidx[k].at[0]], xwb[d], semg[d])

    def wait_gather(k, d):
        pltpu.make_async_copy(A_hbm.at[sdidx[k].at[0]], va[d], semg[d]).wait()
        pltpu.make_async_copy(B_hbm.at[sdidx[k].at[1]], vb[d], semg[d]).wait()
        pltpu.make_async_copy(xw_hbm.at[sdidx[k].at[0]], xwb[d], semg[d]).wait()

    def fire_scatter(k, d):
        pltpu.async_copy(va[d], sh_den.at[sdidx[k].at[1]], sems[d], add=True)
        pltpu.async_copy(xwb[d], sh_msg.at[sdidx[k].at[1]], sems[d], add=True)

    def wait_scatter(k, d):
        pltpu.make_async_copy(va[d], sh_den.at[sdidx[k].at[1]], sems[d]).wait()
        pltpu.make_async_copy(xwb[d], sh_msg.at[sdidx[k].at[1]], sems[d]).wait()

    def compute(d):
        def edge_body(e, carry2):
            ex = jnp.exp(_leaky(va[d][e, :] + vb[d][e, :]))
            va[d][e, :] = ex
            for h in range(H):
                xwb[d][e, pl.ds(h * C, C)] = xwb[d][e, pl.ds(h * C, C)] * ex[h]
            return carry2

        lax.fori_loop(0, CH, edge_body, 0, unroll=False)

    pltpu.sync_copy(zm_hbm.at[pl.ds(s * RPT, RPT)], sh_msg.at[pl.ds(s * RPT, RPT)])
    pltpu.sync_copy(zd_hbm.at[pl.ds(s * RPT, RPT)], sh_den.at[pl.ds(s * RPT, RPT)])
    plsc.subcore_barrier()

    def run_pipeline(nchunk):
        fire_idx(0, 0)
        fire_idx(1, 1)
        wait_idx(0)
        fire_gather(0, 0)

        def group_body(g, carry):
            for cc in range(NIS):
                j = g * NIS + cc
                d = cc % NDS
                wait_gather(cc, d)

                @pl.when(j >= 2)
                def _():
                    wait_scatter((cc - 2) % NIS, (cc + 1) % NDS)

                @pl.when(j < nchunk - 1)
                def _():
                    wait_idx((cc + 1) % NIS)
                    fire_gather((cc + 1) % NIS, (cc + 1) % NDS)

                @pl.when(j < nchunk - 2)
                def _():
                    fire_idx(j + 2, (cc + 2) % NIS)

                compute(d)
                fire_scatter(cc, d)
            return carry

        lax.fori_loop(0, nchunk // NIS, group_body, 0, unroll=False)
        wait_scatter((nchunk - 2) % NIS, (nchunk - 2) % NDS)
        wait_scatter((nchunk - 1) % NIS, (nchunk - 1) % NDS)

    @pl.when(c == 0)
    def _():
        run_pipeline(NCHUNK0)

    @pl.when(c == 1)
    def _():
        run_pipeline(NCHUNK1)

    plsc.subcore_barrier()

    @pl.when(c == 0)
    def _():
        pltpu.sync_copy(sh_msg.at[pl.ds(s * RPT, RPT)], msg0.at[pl.ds(s * RPT, RPT)])
        pltpu.sync_copy(sh_den.at[pl.ds(s * RPT, RPT)], den0.at[pl.ds(s * RPT, RPT)])

    @pl.when(c == 1)
    def _():
        pltpu.sync_copy(sh_msg.at[pl.ds(s * RPT, RPT)], msg1.at[pl.ds(s * RPT, RPT)])
        pltpu.sync_copy(sh_den.at[pl.ds(s * RPT, RPT)], den1.at[pl.ds(s * RPT, RPT)])


@functools.partial(jax.jit, static_argnames=())
def _sc_edge(sdR, A, B, xw, zm, zd):
    f32 = jnp.float32
    return pl.kernel(
        _sc_edge_body,
        out_type=[
            jax.ShapeDtypeStruct((NP, D), f32),
            jax.ShapeDtypeStruct((NP, D), f32),
            jax.ShapeDtypeStruct((NP, 2 * H), f32),
            jax.ShapeDtypeStruct((NP, 2 * H), f32),
        ],
        mesh=plsc.VectorSubcoreMesh(core_axis_name="c", subcore_axis_name="s"),
        compiler_params=pltpu.CompilerParams(use_tc_tiling_on_sc=False),
        scratch_types=(
            [pltpu.VMEM((2, CH), jnp.int32)] * NIS
            + [pltpu.VMEM((CH, 2 * H), f32)] * (2 * NDS)
            + [pltpu.VMEM((CH, D), f32)] * NDS
            + [pltpu.SemaphoreType.DMA] * (NIS + 2 * NDS)
            + [pltpu.VMEM_SHARED((NP, D), f32),
               pltpu.VMEM_SHARED((NP, 2 * H), f32)]
        ),
    )(sdR, A, B, xw, zm, zd)


def kernel(x, edge_index, node_types, proj_W, proj_b,
           gat0_W, gat0_att_src, gat0_att_dst, gat0_bias, ln0_g, ln0_b,
           gat1_W, gat1_att_src, gat1_att_dst, gat1_bias, ln1_g, ln1_b,
           gat2_W, gat2_att_src, gat2_att_dst, gat2_bias, ln2_g, ln2_b,
           pred_W1, pred_b1, pred_W2, pred_b2, pred_W3, pred_b3):
    f32 = jnp.float32
    nb = N // R

    pad = EPAD - E
    src = jnp.concatenate([edge_index[0], jnp.zeros((pad,), jnp.int32)])
    dst = jnp.concatenate([edge_index[1], jnp.full((pad,), N, jnp.int32)])
    sdR = jnp.stack([src.reshape(NROWS, CH), dst.reshape(NROWS, CH)], axis=1)
    zm = jnp.zeros((NP, D), f32)
    zd = jnp.zeros((NP, 2 * H), f32)

    lanes = jnp.arange(D, dtype=jnp.int32)
    heads = jnp.arange(H, dtype=jnp.int32)
    G = (lanes[None, :] // C == heads[:, None]).astype(f32)
    GT = G.T

    atts = [a.reshape(1, D) for a in (gat0_att_src, gat1_att_src, gat2_att_src)]
    attd = [a.reshape(1, D) for a in (gat0_att_dst, gat1_att_dst, gat2_att_dst)]
    biases = [b.reshape(1, D) for b in (gat0_bias, gat1_bias, gat2_bias)]
    lngs = [g.reshape(1, D) for g in (ln0_g, ln1_g, ln2_g)]
    lnbs = [b.reshape(1, D) for b in (ln0_b, ln1_b, ln2_b)]
    Ws = [gat0_W, gat1_W, gat2_W]

    h0, xw0, A0, B0 = pl.pallas_call(
        _pre_body,
        grid=(nb,),
        in_specs=[_row_spec(D), _full_spec((D, D)), _full_spec((1, D)),
                  _full_spec((D, D)), _full_spec((1, D)), _full_spec((1, D)),
                  _full_spec((D, H))],
        out_specs=[_row_spec(D), _row_spec(D), _row_spec(2 * H), _row_spec(2 * H)],
        out_shape=[jax.ShapeDtypeStruct((N, D), f32),
                   jax.ShapeDtypeStruct((N, D), f32),
                   jax.ShapeDtypeStruct((N, 2 * H), f32),
                   jax.ShapeDtypeStruct((N, 2 * H), f32)],
    )(x, proj_W, proj_b.reshape(1, D), Ws[0], atts[0], attd[0], GT)

    h, xw, A, B = h0, xw0, A0, B0
    for l in range(2):
        m0, m1, d0, d1 = _sc_edge(sdR, A, B, xw, zm, zd)
        h, xw, A, B = pl.pallas_call(
            _mid_body,
            grid=(nb,),
            in_specs=[_row_spec(D), _row_spec(D), _row_spec(2 * H), _row_spec(2 * H),
                      _row_spec(2 * H), _row_spec(D), _row_spec(D),
                      _full_spec((1, D)), _full_spec((1, D)), _full_spec((1, D)),
                      _full_spec((H, D)), _full_spec((D, D)),
                      _full_spec((1, D)), _full_spec((1, D)), _full_spec((D, H))],
            out_specs=[_row_spec(D), _row_spec(D), _row_spec(2 * H), _row_spec(2 * H)],
            out_shape=[jax.ShapeDtypeStruct((N, D), f32),
                       jax.ShapeDtypeStruct((N, D), f32),
                       jax.ShapeDtypeStruct((N, 2 * H), f32),
                       jax.ShapeDtypeStruct((N, 2 * H), f32)],
        )(m0, m1, d0, d1, A, xw, h, biases[l], lngs[l], lnbs[l], G,
          Ws[l + 1], atts[l + 1], attd[l + 1], GT)

    m0, m1, d0, d1 = _sc_edge(sdR, A, B, xw, zm, zd)
    out = pl.pallas_call(
        _post_body,
        grid=(nb,),
        in_specs=[_row_spec(D), _row_spec(D), _row_spec(2 * H), _row_spec(2 * H),
                  _row_spec(2 * H), _row_spec(D), _row_spec(D),
                  _full_spec((1, D)), _full_spec((1, D)), _full_spec((1, D)),
                  _full_spec((H, D)),
                  _full_spec((D, D)), _full_spec((1, D)),
                  _full_spec((D, D // 2)), _full_spec((1, D // 2)),
                  _full_spec((D // 2, 1)), _full_spec((1, 1))],
        out_specs=[_row_spec(1)],
        out_shape=[jax.ShapeDtypeStruct((N, 1), f32)],
    )(m0, m1, d0, d1, A, xw, h, biases[2], lngs[2], lnbs[2], G,
      pred_W1, pred_b1.reshape(1, D), pred_W2, pred_b2.reshape(1, D // 2),
      pred_W3, pred_b3.reshape(1, 1))[0]
    return out


# speedup vs baseline: 1.0145x; 1.0145x over previous
import functools

import jax
import jax.numpy as jnp
from jax import lax
from jax.experimental import pallas as pl
from jax.experimental.pallas import tpu as pltpu
from jax.experimental.pallas import tpu_sc as plsc

N = 10000
D = 128
H = 8
C = 16
E = 320000

NW = 32
CH = 80
NCHUNK0 = 150
NCHUNK1 = 102
NROWS = 16 * (NCHUNK0 + NCHUNK1)
EPAD = NROWS * CH
NP = 10016
RPT = NP // 16
NDS = 3
NIS = 6
R = 1000


def _leaky(x):
    return jnp.maximum(x, 0.0) + 0.2 * jnp.minimum(x, 0.0)


def _pre_body(x_ref, pW_ref, pb_ref, W_ref, ats_ref, atd_ref, GT_ref,
              h_ref, xw_ref, A_ref, B_ref):
    xb = x_ref[...]
    h = jnp.dot(xb, pW_ref[...], preferred_element_type=jnp.float32) + pb_ref[...]
    xw = jnp.dot(h, W_ref[...], preferred_element_type=jnp.float32)
    a_s = jnp.dot(xw * ats_ref[...], GT_ref[...], preferred_element_type=jnp.float32)
    a_d = jnp.dot(xw * atd_ref[...], GT_ref[...], preferred_element_type=jnp.float32)
    h_ref[...] = h
    xw_ref[...] = xw
    A_ref[...] = jnp.concatenate([a_s, a_d], axis=1)
    B_ref[...] = jnp.concatenate([a_d, a_s], axis=1)


def _combine(m0_ref, m1_ref, d0_ref, d1_ref, A_ref, xw_ref, hres_ref,
             bias_ref, g_ref, b_ref, G_ref):
    m = m0_ref[...] + m1_ref[...]
    d16 = d0_ref[...] + d1_ref[...]
    Ab = A_ref[...]
    es = jnp.exp(_leaky(Ab[:, :H] + Ab[:, H:]))
    xwb = xw_ref[...]
    G = G_ref[...]
    dtot = d16[:, :H] + es + 1e-16
    mtot = m + jnp.dot(es, G, preferred_element_type=jnp.float32) * xwb
    out = mtot / jnp.dot(dtot, G, preferred_element_type=jnp.float32)
    y = jnp.maximum(out + bias_ref[...], 0.0) + hres_ref[...]
    mu = jnp.mean(y, axis=1, keepdims=True)
    yc = y - mu
    var = jnp.mean(yc * yc, axis=1, keepdims=True)
    return yc * lax.rsqrt(var + 1e-5) * g_ref[...] + b_ref[...]


def _mid_body(m0_ref, m1_ref, d0_ref, d1_ref, A_ref, xw_ref, hres_ref,
              bias_ref, g_ref, b_ref, G_ref, W_ref, ats_ref, atd_ref, GT_ref,
              h_ref, xw_out_ref, A_out_ref, B_out_ref):
    hn = _combine(m0_ref, m1_ref, d0_ref, d1_ref, A_ref, xw_ref, hres_ref,
                  bias_ref, g_ref, b_ref, G_ref)
    xwn = jnp.dot(hn, W_ref[...], preferred_element_type=jnp.float32)
    a_s = jnp.dot(xwn * ats_ref[...], GT_ref[...], preferred_element_type=jnp.float32)
    a_d = jnp.dot(xwn * atd_ref[...], GT_ref[...], preferred_element_type=jnp.float32)
    h_ref[...] = hn
    xw_out_ref[...] = xwn
    A_out_ref[...] = jnp.concatenate([a_s, a_d], axis=1)
    B_out_ref[...] = jnp.concatenate([a_d, a_s], axis=1)


def _post_body(m0_ref, m1_ref, d0_ref, d1_ref, A_ref, xw_ref, hres_ref,
               bias_ref, g_ref, b_ref, G_ref,
               W1_ref, b1_ref, W2_ref, b2_ref, W3_ref, b3_ref, out_ref):
    hn = _combine(m0_ref, m1_ref, d0_ref, d1_ref, A_ref, xw_ref, hres_ref,
                  bias_ref, g_ref, b_ref, G_ref)
    y1 = jnp.maximum(jnp.dot(hn, W1_ref[...], preferred_element_type=jnp.float32)
                     + b1_ref[...], 0.0)
    y2 = jnp.maximum(jnp.dot(y1, W2_ref[...], preferred_element_type=jnp.float32)
                     + b2_ref[...], 0.0)
    out_ref[...] = (jnp.dot(y2, W3_ref[...], preferred_element_type=jnp.float32)
                    + b3_ref[...])


def _row_spec(cols):
    return pl.BlockSpec((R, cols), lambda i: (i, 0))


def _full_spec(shape):
    nd = len(shape)
    return pl.BlockSpec(shape, lambda i: (0,) * nd)


def _sc_edge_body(src_hbm, dst_hbm, A_hbm, B_hbm, xw_hbm, zm_hbm, zd_hbm,
                  msg0, msg1, den0, den1, *scr):
    sidx = scr[0:NIS]
    didx = scr[NIS:2 * NIS]
    va = scr[2 * NIS:2 * NIS + NDS]
    vb = scr[2 * NIS + NDS:2 * NIS + 2 * NDS]
    xwb = scr[2 * NIS + 2 * NDS:2 * NIS + 3 * NDS]
    semi = scr[2 * NIS + 3 * NDS:3 * NIS + 3 * NDS]
    semg = scr[3 * NIS + 3 * NDS:3 * NIS + 4 * NDS]
    sems = scr[3 * NIS + 4 * NDS:3 * NIS + 5 * NDS]
    sh_msg, sh_den = scr[-2], scr[-1]

    c = lax.axis_index("c")
    s = lax.axis_index("s")
    rowbase = jnp.where(c == 0, s * NCHUNK0, 16 * NCHUNK0 + s * NCHUNK1)

    def fire_idx(j, k):
        pltpu.async_copy(src_hbm.at[rowbase + j], sidx[k], semi[k])
        pltpu.async_copy(dst_hbm.at[rowbase + j], didx[k], semi[k])

    def wait_idx(k):
        pltpu.make_async_copy(src_hbm.at[rowbase], sidx[k], semi[k]).wait()
        pltpu.make_async_copy(dst_hbm.at[rowbase], didx[k], semi[k]).wait()

    def fire_gather(k, d):
        pltpu.async_copy(A_hbm.at[sidx[k]], va[d], semg[d])
        pltpu.async_copy(B_hbm.at[didx[k]], vb[d], semg[d])
        pltpu.async_copy(xw_hbm.at[sidx[k]], xwb[d], semg[d])

    def wait_gather(k, d):
        pltpu.make_async_copy(A_hbm.at[sidx[k]], va[d], semg[d]).wait()
        pltpu.make_async_copy(B_hbm.at[didx[k]], vb[d], semg[d]).wait()
        pltpu.make_async_copy(xw_hbm.at[sidx[k]], xwb[d], semg[d]).wait()

    def fire_scatter(k, d):
        pltpu.async_copy(va[d], sh_den.at[didx[k]], sems[d], add=True)
        pltpu.async_copy(xwb[d], sh_msg.at[didx[k]], sems[d], add=True)

    def wait_scatter(k, d):
        pltpu.make_async_copy(va[d], sh_den.at[didx[k]], sems[d]).wait()
        pltpu.make_async_copy(xwb[d], sh_msg.at[didx[k]], sems[d]).wait()

    def compute(d):
        def edge_body(e, carry2):
            ex = jnp.exp(_leaky(va[d][e, :] + vb[d][e, :]))
            va[d][e, :] = ex
            for h in range(H):
                xwb[d][e, pl.ds(h * C, C)] = xwb[d][e, pl.ds(h * C, C)] * ex[h]
            return carry2

        lax.fori_loop(0, CH, edge_body, 0, unroll=False)

    pltpu.sync_copy(zm_hbm.at[pl.ds(s * RPT, RPT)], sh_msg.at[pl.ds(s * RPT, RPT)])
    pltpu.sync_copy(zd_hbm.at[pl.ds(s * RPT, RPT)], sh_den.at[pl.ds(s * RPT, RPT)])
    plsc.subcore_barrier()

    def run_pipeline(nchunk):
        fire_idx(0, 0)
        fire_idx(1, 1)
        wait_idx(0)
        fire_gather(0, 0)

        def group_body(g, carry):
            for cc in range(NIS):
                j = g * NIS + cc
                d = cc % NDS
                wait_gather(cc, d)

                @pl.when(j >= 2)
                def _():
                    wait_scatter((cc - 2) % NIS, (cc + 1) % NDS)

                @pl.when(j < nchunk - 1)
                def _():
                    wait_idx((cc + 1) % NIS)
                    fire_gather((cc + 1) % NIS, (cc + 1) % NDS)

                @pl.when(j < nchunk - 2)
                def _():
                    fire_idx(j + 2, (cc + 2) % NIS)

                compute(d)
                fire_scatter(cc, d)
            return carry

        lax.fori_loop(0, nchunk // NIS, group_body, 0, unroll=False)
        wait_scatter((nchunk - 2) % NIS, (nchunk - 2) % NDS)
        wait_scatter((nchunk - 1) % NIS, (nchunk - 1) % NDS)

    @pl.when(c == 0)
    def _():
        run_pipeline(NCHUNK0)

    @pl.when(c == 1)
    def _():
        run_pipeline(NCHUNK1)

    plsc.subcore_barrier()

    @pl.when(c == 0)
    def _():
        pltpu.sync_copy(sh_msg.at[pl.ds(s * RPT, RPT)], msg0.at[pl.ds(s * RPT, RPT)])
        pltpu.sync_copy(sh_den.at[pl.ds(s * RPT, RPT)], den0.at[pl.ds(s * RPT, RPT)])

    @pl.when(c == 1)
    def _():
        pltpu.sync_copy(sh_msg.at[pl.ds(s * RPT, RPT)], msg1.at[pl.ds(s * RPT, RPT)])
        pltpu.sync_copy(sh_den.at[pl.ds(s * RPT, RPT)], den1.at[pl.ds(s * RPT, RPT)])


@functools.partial(jax.jit, static_argnames=())
def _sc_edge(srcR, dstR, A, B, xw, zm, zd):
    f32 = jnp.float32
    return pl.kernel(
        _sc_edge_body,
        out_type=[
            jax.ShapeDtypeStruct((NP, D), f32),
            jax.ShapeDtypeStruct((NP, D), f32),
            jax.ShapeDtypeStruct((NP, 2 * H), f32),
            jax.ShapeDtypeStruct((NP, 2 * H), f32),
        ],
        mesh=plsc.VectorSubcoreMesh(core_axis_name="c", subcore_axis_name="s"),
        compiler_params=pltpu.CompilerParams(use_tc_tiling_on_sc=False),
        scratch_types=(
            [pltpu.VMEM((CH,), jnp.int32)] * (2 * NIS)
            + [pltpu.VMEM((CH, 2 * H), f32)] * (2 * NDS)
            + [pltpu.VMEM((CH, D), f32)] * NDS
            + [pltpu.SemaphoreType.DMA] * (NIS + 2 * NDS)
            + [pltpu.VMEM_SHARED((NP, D), f32),
               pltpu.VMEM_SHARED((NP, 2 * H), f32)]
        ),
    )(srcR, dstR, A, B, xw, zm, zd)


def kernel(x, edge_index, node_types, proj_W, proj_b,
           gat0_W, gat0_att_src, gat0_att_dst, gat0_bias, ln0_g, ln0_b,
           gat1_W, gat1_att_src, gat1_att_dst, gat1_bias, ln1_g, ln1_b,
           gat2_W, gat2_att_src, gat2_att_dst, gat2_bias, ln2_g, ln2_b,
           pred_W1, pred_b1, pred_W2, pred_b2, pred_W3, pred_b3):
    f32 = jnp.float32
    nb = N // R

    pad = EPAD - E
    src = jnp.concatenate([edge_index[0], jnp.zeros((pad,), jnp.int32)])
    dst = jnp.concatenate([edge_index[1], jnp.full((pad,), N, jnp.int32)])
    srcR = src.reshape(NROWS, CH)
    dstR = dst.reshape(NROWS, CH)
    zm = jnp.zeros((NP, D), f32)
    zd = jnp.zeros((NP, 2 * H), f32)

    lanes = jnp.arange(D, dtype=jnp.int32)
    heads = jnp.arange(H, dtype=jnp.int32)
    G = (lanes[None, :] // C == heads[:, None]).astype(f32)
    GT = G.T

    atts = [a.reshape(1, D) for a in (gat0_att_src, gat1_att_src, gat2_att_src)]
    attd = [a.reshape(1, D) for a in (gat0_att_dst, gat1_att_dst, gat2_att_dst)]
    biases = [b.reshape(1, D) for b in (gat0_bias, gat1_bias, gat2_bias)]
    lngs = [g.reshape(1, D) for g in (ln0_g, ln1_g, ln2_g)]
    lnbs = [b.reshape(1, D) for b in (ln0_b, ln1_b, ln2_b)]
    Ws = [gat0_W, gat1_W, gat2_W]

    h0, xw0, A0, B0 = pl.pallas_call(
        _pre_body,
        grid=(nb,),
        in_specs=[_row_spec(D), _full_spec((D, D)), _full_spec((1, D)),
                  _full_spec((D, D)), _full_spec((1, D)), _full_spec((1, D)),
                  _full_spec((D, H))],
        out_specs=[_row_spec(D), _row_spec(D), _row_spec(2 * H), _row_spec(2 * H)],
        out_shape=[jax.ShapeDtypeStruct((N, D), f32),
                   jax.ShapeDtypeStruct((N, D), f32),
                   jax.ShapeDtypeStruct((N, 2 * H), f32),
                   jax.ShapeDtypeStruct((N, 2 * H), f32)],
    )(x, proj_W, proj_b.reshape(1, D), Ws[0], atts[0], attd[0], GT)

    h, xw, A, B = h0, xw0, A0, B0
    for l in range(2):
        m0, m1, d0, d1 = _sc_edge(srcR, dstR, A, B, xw, zm, zd)
        h, xw, A, B = pl.pallas_call(
            _mid_body,
            grid=(nb,),
            in_specs=[_row_spec(D), _row_spec(D), _row_spec(2 * H), _row_spec(2 * H),
                      _row_spec(2 * H), _row_spec(D), _row_spec(D),
                      _full_spec((1, D)), _full_spec((1, D)), _full_spec((1, D)),
                      _full_spec((H, D)), _full_spec((D, D)),
                      _full_spec((1, D)), _full_spec((1, D)), _full_spec((D, H))],
            out_specs=[_row_spec(D), _row_spec(D), _row_spec(2 * H), _row_spec(2 * H)],
            out_shape=[jax.ShapeDtypeStruct((N, D), f32),
                       jax.ShapeDtypeStruct((N, D), f32),
                       jax.ShapeDtypeStruct((N, 2 * H), f32),
                       jax.ShapeDtypeStruct((N, 2 * H), f32)],
        )(m0, m1, d0, d1, A, xw, h, biases[l], lngs[l], lnbs[l], G,
          Ws[l + 1], atts[l + 1], attd[l + 1], GT)

    m0, m1, d0, d1 = _sc_edge(srcR, dstR, A, B, xw, zm, zd)
    out = pl.pallas_call(
        _post_body,
        grid=(nb,),
        in_specs=[_row_spec(D), _row_spec(D), _row_spec(2 * H), _row_spec(2 * H),
                  _row_spec(2 * H), _row_spec(D), _row_spec(D),
                  _full_spec((1, D)), _full_spec((1, D)), _full_spec((1, D)),
                  _full_spec((H, D)),
                  _full_spec((D, D)), _full_spec((1, D)),
                  _full_spec((D, D // 2)), _full_spec((1, D // 2)),
                  _full_spec((D // 2, 1)), _full_spec((1, 1))],
        out_specs=[_row_spec(1)],
        out_shape=[jax.ShapeDtypeStruct((N, 1), f32)],
    )(m0, m1, d0, d1, A, xw, h, biases[2], lngs[2], lnbs[2], G,
      pred_W1, pred_b1.reshape(1, D), pred_W2, pred_b2.reshape(1, D // 2),
      pred_W3, pred_b3.reshape(1, 1))[0]
    return out
